# baseline (speedup 1.0000x reference)
import functools

import jax
import jax.numpy as jnp
from jax.experimental import pallas as pl
from jax.experimental.pallas import tpu as pltpu
from jax.experimental.pallas import tpu_sc as plsc

B, N, NBR = 4, 2500, 32
F, G = 128, 64

TILE_N = 512
NT = (N + TILE_N - 1) // TILE_N
GATHER_WINDOW = 128


def _init_body(feat_ref, w_ref, out_ref):
    out_ref[...] = jnp.dot(feat_ref[...], w_ref[...],
                           preferred_element_type=jnp.float32)


def _init_features(features_flat, W_init):
    return pl.pallas_call(
        _init_body,
        out_shape=jax.ShapeDtypeStruct((B * N, F), jnp.float32),
    )(features_flat, W_init)


def _sc_gather(table, idx_flat):
    m = idx_flat.shape[1]
    mesh = plsc.VectorSubcoreMesh(core_axis_name="c", subcore_axis_name="s")

    @functools.partial(
        pl.kernel,
        out_type=jax.ShapeDtypeStruct((m, F), jnp.float32),
        mesh=mesh,
    )
    def k(table_hbm, idx_hbm, out_hbm):
        def body(i_vmem, o_vmem):
            def inner(sems):
                h1 = pltpu.async_copy(
                    table_hbm.at[i_vmem.at[0]],
                    o_vmem.at[pl.ds(0, GATHER_WINDOW)], sems.at[0])
                h2 = pltpu.async_copy(
                    table_hbm.at[i_vmem.at[1]],
                    o_vmem.at[pl.ds(GATHER_WINDOW, GATHER_WINDOW)], sems.at[1])
                h1.wait()
                h2.wait()
            pl.run_scoped(inner, pltpu.SemaphoreType.DMA((2,)))

        pltpu.emit_pipeline(
            body,
            grid=(m // (2 * GATHER_WINDOW),),
            in_specs=[pl.BlockSpec((2, GATHER_WINDOW), lambda i: (i, 0))],
            out_specs=[pl.BlockSpec((2 * GATHER_WINDOW, F), lambda i: (i, 0))],
            core_axis_name=("c", "s"),
            dimension_semantics=(pltpu.PARALLEL,),
        )(idx_hbm, out_hbm)

    return k(table, idx_flat.reshape(m // GATHER_WINDOW, GATHER_WINDOW))


def _fused_body(rbf_ref, gath_ref, w1_ref, b1_ref, w2_ref, b2_ref, v_ref,
                wo1_ref, bo1_ref, wo2_ref, bo2_ref, out_ref, attn_ref):
    rbf = rbf_ref[...].reshape(TILE_N * NBR, G).astype(jnp.bfloat16)
    h = jnp.tanh(jnp.dot(rbf, w1_ref[...].astype(jnp.bfloat16),
                         preferred_element_type=jnp.float32) + b1_ref[...])
    filt = jnp.dot(h.astype(jnp.bfloat16), w2_ref[...].astype(jnp.bfloat16),
                   preferred_element_type=jnp.float32) + b2_ref[...]
    conv = gath_ref[...].reshape(TILE_N * NBR, F) * filt
    conv3 = conv.reshape(TILE_N, NBR, F)
    logits = jnp.sum(conv3 * v_ref[...].reshape(1, 1, F), axis=-1,
                     keepdims=True)
    m = jnp.max(logits, axis=1, keepdims=True)
    e = jnp.exp(logits - m)
    attn3 = e / jnp.sum(e, axis=1, keepdims=True)
    attn_ref[...] = attn3.reshape(1, TILE_N, NBR)
    agg = jnp.sum(conv3 * attn3, axis=1)
    out = jnp.dot(jnp.tanh(jnp.dot(agg.astype(jnp.bfloat16),
                                   wo1_ref[...].astype(jnp.bfloat16),
                                   preferred_element_type=jnp.float32)
                           + bo1_ref[...]).astype(jnp.bfloat16),
                  wo2_ref[...].astype(jnp.bfloat16),
                  preferred_element_type=jnp.float32) + bo2_ref[...]
    out_ref[...] = out.reshape(1, TILE_N, F)


def _fused(rbf, gath, W1, b1, W2, b2, v_row, Wo1, bo1, Wo2, bo2):
    full = lambda shape: pl.BlockSpec(shape, lambda b, i: tuple(0 for _ in shape))
    return pl.pallas_call(
        _fused_body,
        grid=(B, NT),
        in_specs=[
            pl.BlockSpec((1, TILE_N, NBR, G), lambda b, i: (b, i, 0, 0)),
            pl.BlockSpec((1, TILE_N, NBR, F), lambda b, i: (b, i, 0, 0)),
            full((G, F)), full((1, F)), full((F, F)), full((1, F)),
            full((1, F)),
            full((F, F)), full((1, F)), full((F, F)), full((1, F)),
        ],
        out_specs=[
            pl.BlockSpec((1, TILE_N, F), lambda b, i: (b, i, 0)),
            pl.BlockSpec((1, TILE_N, NBR), lambda b, i: (b, i, 0)),
        ],
        out_shape=[
            jax.ShapeDtypeStruct((B, N, F), jnp.float32),
            jax.ShapeDtypeStruct((B, N, NBR), jnp.float32),
        ],
    )(rbf, gath, W1, b1, W2, b2, v_row, Wo1, bo1, Wo2, bo2)


def kernel(features, rbf_expansion, neighbor_list, W_init, W1, b1, W2, b2,
           nbr_filter, Wo1, bo1, Wo2, bo2):
    init = _init_features(features.reshape(B * N, F), W_init)
    b1r, b2r = b1.reshape(1, F), b2.reshape(1, F)
    bo1r, bo2r = bo1.reshape(1, F), bo2.reshape(1, F)
    v_row = nbr_filter.reshape(1, F)
    idx = (neighbor_list
           + (jnp.arange(B, dtype=jnp.int32) * N)[:, None, None])
    gath = _sc_gather(init, idx.reshape(1, B * N * NBR))
    out, attn = _fused(rbf_expansion, gath.reshape(B, N, NBR, F),
                       W1, b1r, W2, b2r, v_row, Wo1, bo1r, Wo2, bo2r)
    return out, attn


# speedup vs baseline: 1.0178x; 1.0178x over previous
import functools

import jax
import jax.numpy as jnp
from jax.experimental import pallas as pl
from jax.experimental.pallas import tpu as pltpu
from jax.experimental.pallas import tpu_sc as plsc

B, N, NBR = 4, 2500, 32
F, G = 128, 64

TILE_N = 512
NT = (N + TILE_N - 1) // TILE_N
GATHER_WINDOW = 128


def _init_body(feat_ref, w_ref, out_ref):
    out_ref[...] = jnp.dot(feat_ref[...], w_ref[...],
                           preferred_element_type=jnp.float32)


def _init_features(features_flat, W_init):
    return pl.pallas_call(
        _init_body,
        out_shape=jax.ShapeDtypeStruct((B * N, F), jnp.float32),
    )(features_flat, W_init)


def _sc_gather(table, idx_flat):
    m = idx_flat.shape[1]
    mesh = plsc.VectorSubcoreMesh(core_axis_name="c", subcore_axis_name="s")

    @functools.partial(
        pl.kernel,
        out_type=jax.ShapeDtypeStruct((m, F), jnp.float32),
        mesh=mesh,
    )
    def k(table_hbm, idx_hbm, out_hbm):
        def body(i_vmem, o_vmem):
            pltpu.sync_copy(table_hbm.at[i_vmem.at[0]], o_vmem)

        pltpu.emit_pipeline(
            body,
            grid=(m // GATHER_WINDOW,),
            in_specs=[pl.BlockSpec((1, GATHER_WINDOW), lambda i: (0, i))],
            out_specs=[pl.BlockSpec((GATHER_WINDOW, F), lambda i: (i, 0))],
            core_axis_name=("c", "s"),
            dimension_semantics=(pltpu.PARALLEL,),
        )(idx_hbm, out_hbm)

    return k(table, idx_flat)


def _fused_body(rbf_ref, gath_ref, w1_ref, b1_ref, w2_ref, b2_ref, v_ref,
                wo1_ref, bo1_ref, wo2_ref, bo2_ref, out_ref, attn_ref):
    rbf = rbf_ref[...].reshape(TILE_N * NBR, G).astype(jnp.bfloat16)
    h = jnp.tanh(jnp.dot(rbf, w1_ref[...].astype(jnp.bfloat16),
                         preferred_element_type=jnp.float32) + b1_ref[...])
    filt = jnp.dot(h.astype(jnp.bfloat16), w2_ref[...].astype(jnp.bfloat16),
                   preferred_element_type=jnp.float32) + b2_ref[...]
    conv = gath_ref[...].reshape(TILE_N * NBR, F) * filt
    conv3 = conv.reshape(TILE_N, NBR, F)
    logits = jnp.sum(conv3 * v_ref[...].reshape(1, 1, F), axis=-1,
                     keepdims=True)
    m = jnp.max(logits, axis=1, keepdims=True)
    e = jnp.exp(logits - m)
    attn3 = e / jnp.sum(e, axis=1, keepdims=True)
    attn_ref[...] = attn3.reshape(1, TILE_N, NBR)
    agg = jnp.sum(conv3 * attn3, axis=1)
    out = jnp.dot(jnp.tanh(jnp.dot(agg.astype(jnp.bfloat16),
                                   wo1_ref[...].astype(jnp.bfloat16),
                                   preferred_element_type=jnp.float32)
                           + bo1_ref[...]).astype(jnp.bfloat16),
                  wo2_ref[...].astype(jnp.bfloat16),
                  preferred_element_type=jnp.float32) + bo2_ref[...]
    out_ref[...] = out.reshape(1, TILE_N, F)


def _fused(rbf, gath, W1, b1, W2, b2, v_row, Wo1, bo1, Wo2, bo2):
    full = lambda shape: pl.BlockSpec(shape, lambda b, i: tuple(0 for _ in shape))
    return pl.pallas_call(
        _fused_body,
        grid=(B, NT),
        in_specs=[
            pl.BlockSpec((1, TILE_N, NBR, G), lambda b, i: (b, i, 0, 0)),
            pl.BlockSpec((1, TILE_N, NBR, F), lambda b, i: (b, i, 0, 0)),
            full((G, F)), full((1, F)), full((F, F)), full((1, F)),
            full((1, F)),
            full((F, F)), full((1, F)), full((F, F)), full((1, F)),
        ],
        out_specs=[
            pl.BlockSpec((1, TILE_N, F), lambda b, i: (b, i, 0)),
            pl.BlockSpec((1, TILE_N, NBR), lambda b, i: (b, i, 0)),
        ],
        out_shape=[
            jax.ShapeDtypeStruct((B, N, F), jnp.float32),
            jax.ShapeDtypeStruct((B, N, NBR), jnp.float32),
        ],
    )(rbf, gath, W1, b1, W2, b2, v_row, Wo1, bo1, Wo2, bo2)


def kernel(features, rbf_expansion, neighbor_list, W_init, W1, b1, W2, b2,
           nbr_filter, Wo1, bo1, Wo2, bo2):
    init = _init_features(features.reshape(B * N, F), W_init)
    b1r, b2r = b1.reshape(1, F), b2.reshape(1, F)
    bo1r, bo2r = bo1.reshape(1, F), bo2.reshape(1, F)
    v_row = nbr_filter.reshape(1, F)
    idx = (neighbor_list
           + (jnp.arange(B, dtype=jnp.int32) * N)[:, None, None])
    gath = _sc_gather(init, idx.reshape(1, B * N * NBR))
    out, attn = _fused(rbf_expansion, gath.reshape(B, N, NBR, F),
                       W1, b1r, W2, b2r, v_row, Wo1, bo1r, Wo2, bo2r)
    return out, attn
